# Initial kernel scaffold; baseline (speedup 1.0000x reference)
import functools

import jax
import jax.numpy as jnp
from jax import lax
from jax.experimental import pallas as pl
from jax.experimental.pallas import tpu as pltpu
from jax.experimental.pallas import tpu_sc as plsc

EPS = 0.0001
LN_EPS = 1e-5


def _softplus(v):
    return jnp.maximum(v, 0.0) + jnp.log1p(jnp.exp(-jnp.abs(v)))


def _layer_norm(v, w, b):
    mu = jnp.mean(v, axis=-1, keepdims=True)
    d = v - mu
    var = jnp.mean(d * d, axis=-1, keepdims=True)
    return d * lax.rsqrt(var + LN_EPS) * w + b


def _pre_body(x_ref, wfc, bfc, wrate, wrb1, brb1, wrb2, brb2, lnw, lnb,
              xf_ref, rate_ref, gamma_ref):
    xb = x_ref[...]
    xf_ref[...] = jnp.dot(xb, wfc[...], preferred_element_type=jnp.float32) + bfc[...]
    rate_ref[...] = _softplus(
        jnp.dot(xb, wrate[...], preferred_element_type=jnp.float32))
    h = _softplus(
        jnp.dot(xb, wrb1[...], preferred_element_type=jnp.float32) + brb1[...])
    g = jnp.dot(h, wrb2[...], preferred_element_type=jnp.float32) + brb2[...]
    gamma_ref[...] = _layer_norm(g, lnw[...], lnb[...])


def _dense_pre(x, wfc_t, bfc, wrate_t, wrb1_t, brb1, wrb2_t, brb2, lnw, lnb):
    n, d = x.shape
    blk = 1000
    grid = (n // blk,)
    row_spec = pl.BlockSpec((blk, d), lambda i: (i, 0))
    full = pl.BlockSpec((d, d), lambda i: (0, 0))
    vec = pl.BlockSpec((1, d), lambda i: (0, 0))
    out_sds = jax.ShapeDtypeStruct((n, d), jnp.float32)
    return pl.pallas_call(
        _pre_body,
        grid=grid,
        in_specs=[row_spec, full, vec, full, full, vec, full, vec, vec, vec],
        out_specs=[row_spec, row_spec, row_spec],
        out_shape=[out_sds, out_sds, out_sds],
    )(x, wfc_t, bfc, wrate_t, wrb1_t, brb1, wrb2_t, brb2, lnw, lnb)


def _sc_aggregate(xf, row, col):
    n, d = xf.shape
    e = row.shape[0]
    info = plsc.get_sparse_core_info()
    nc, ns = info.num_cores, info.num_subcores
    nw = nc * ns
    chunk = 80
    e_per_tile = e // nw
    n_chunks = e_per_tile // chunk
    rows_per_tile = n // ns
    zrows = 125
    n_zcopies = rows_per_tile // zrows

    mesh = plsc.VectorSubcoreMesh(core_axis_name="c", subcore_axis_name="s")

    @functools.partial(
        pl.kernel,
        out_type=(
            jax.ShapeDtypeStruct((nc, n, d), jnp.float32),
            jax.ShapeDtypeStruct((nc, n, 16), jnp.float32),
        ),
        mesh=mesh,
        scratch_types=[
            pltpu.VMEM((chunk,), jnp.int32),
            pltpu.VMEM((chunk,), jnp.int32),
            pltpu.VMEM((chunk, d), jnp.float32),
            pltpu.VMEM((chunk, 16), jnp.float32),
            pltpu.VMEM((zrows, d), jnp.float32),
            pltpu.VMEM((zrows, 16), jnp.float32),
            pltpu.VMEM_SHARED((n, d), jnp.float32),
            pltpu.VMEM_SHARED((n, 16), jnp.float32),
            pltpu.SemaphoreType.DMA,
        ],
    )
    def agg_kernel(xf_hbm, row_hbm, col_hbm, s_out, cnt_out,
                   ridx, cidx, rows, ones, zs, zc, s_sh, c_sh, sem):
        c = lax.axis_index("c")
        s = lax.axis_index("s")
        wid = c * ns + s
        z16 = jnp.zeros((16,), jnp.float32)
        one16 = jnp.ones((16,), jnp.float32)

        def init_z(i, carry):
            for k in range(d // 16):
                zs[i, pl.ds(k * 16, 16)] = z16
            zc[i] = z16
            return carry

        lax.fori_loop(0, zrows, init_z, 0)

        def init_ones(i, carry):
            ones[i] = one16
            return carry

        lax.fori_loop(0, chunk, init_ones, 0)

        base_rows = s * rows_per_tile

        def zero_copy(i, carry):
            pltpu.sync_copy(zs, s_sh.at[pl.ds(base_rows + i * zrows, zrows)])
            pltpu.sync_copy(zc, c_sh.at[pl.ds(base_rows + i * zrows, zrows)])
            return carry

        lax.fori_loop(0, n_zcopies, zero_copy, 0)
        plsc.subcore_barrier()

        e0 = wid * e_per_tile

        def body(j, carry):
            base = e0 + j * chunk
            pltpu.sync_copy(row_hbm.at[pl.ds(base, chunk)], ridx)
            pltpu.sync_copy(col_hbm.at[pl.ds(base, chunk)], cidx)
            pltpu.async_copy(xf_hbm.at[cidx], rows, sem).wait()
            pltpu.sync_copy(rows, s_sh.at[ridx], add=True)
            pltpu.sync_copy(ones, c_sh.at[ridx], add=True)
            return carry

        lax.fori_loop(0, n_chunks, body, 0)
        plsc.subcore_barrier()

        pltpu.sync_copy(s_sh.at[pl.ds(base_rows, rows_per_tile)],
                        s_out.at[c, pl.ds(base_rows, rows_per_tile)])
        pltpu.sync_copy(c_sh.at[pl.ds(base_rows, rows_per_tile)],
                        cnt_out.at[c, pl.ds(base_rows, rows_per_tile)])

    return agg_kernel(xf, row, col)


def _post_body(xf_ref, rate_ref, gamma_ref, s_ref, cnt_ref, deg_ref, lnw, lnb,
               out_ref):
    xf = xf_ref[...]
    rate = rate_ref[...]
    ssum = s_ref[0] + s_ref[1]
    cnt = cnt_ref[0, :, 0:1] + cnt_ref[1, :, 0:1]
    agg = cnt * xf + ssum
    denom = 1.0 + rate * deg_ref[...] + EPS
    out = (rate * agg + gamma_ref[...]) / denom - xf
    out_ref[...] = _layer_norm(out, lnw[...], lnb[...])


def _dense_post(xf, rate, gamma, s_part, cnt_part, deg_col, lnw, lnb):
    n, d = xf.shape
    blk = 1000
    grid = (n // blk,)
    row_spec = pl.BlockSpec((blk, d), lambda i: (i, 0))
    s_spec = pl.BlockSpec((2, blk, d), lambda i: (0, i, 0))
    cnt_spec = pl.BlockSpec((2, blk, 16), lambda i: (0, i, 0))
    deg_spec = pl.BlockSpec((blk, 1), lambda i: (i, 0))
    vec = pl.BlockSpec((1, d), lambda i: (0, 0))
    return pl.pallas_call(
        _post_body,
        grid=grid,
        in_specs=[row_spec, row_spec, row_spec, s_spec, cnt_spec, deg_spec,
                  vec, vec],
        out_specs=row_spec,
        out_shape=jax.ShapeDtypeStruct((n, d), jnp.float32),
    )(xf, rate, gamma, s_part, cnt_part, deg_col, lnw, lnb)


def kernel(x, edge_index, degree, W_fc, b_fc, W_rate, W_rb1, b_rb1, W_rb2,
           b_rb2, ln_rb_w, ln_rb_b, ln_w, ln_b):
    n, d = x.shape
    row = edge_index[0]
    col = edge_index[1]
    xf, rate, gamma = _dense_pre(
        x, W_fc.T, b_fc.reshape(1, d), W_rate.T, W_rb1.T, b_rb1.reshape(1, d),
        W_rb2.T, b_rb2.reshape(1, d), ln_rb_w.reshape(1, d),
        ln_rb_b.reshape(1, d))
    s_part, cnt_part = _sc_aggregate(xf, row, col)
    return _dense_post(xf, rate, gamma, s_part, cnt_part,
                       degree.reshape(n, 1), ln_w.reshape(1, d),
                       ln_b.reshape(1, d))


# speedup vs baseline: 4.2856x; 4.2856x over previous
import functools

import jax
import jax.numpy as jnp
from jax import lax
from jax.experimental import pallas as pl
from jax.experimental.pallas import tpu as pltpu
from jax.experimental.pallas import tpu_sc as plsc

EPS = 0.0001
LN_EPS = 1e-5


def _softplus(v):
    return jnp.maximum(v, 0.0) + jnp.log1p(jnp.exp(-jnp.abs(v)))


def _layer_norm(v, w, b):
    mu = jnp.mean(v, axis=-1, keepdims=True)
    d = v - mu
    var = jnp.mean(d * d, axis=-1, keepdims=True)
    return d * lax.rsqrt(var + LN_EPS) * w + b


def _pre_body(x_ref, wfc, bfc, wrate, wrb1, brb1, wrb2, brb2, lnw, lnb,
              xf_ref, rate_ref, gamma_ref):
    xb = x_ref[...]
    xf_ref[...] = jnp.dot(xb, wfc[...], preferred_element_type=jnp.float32) + bfc[...]
    rate_ref[...] = _softplus(
        jnp.dot(xb, wrate[...], preferred_element_type=jnp.float32))
    h = _softplus(
        jnp.dot(xb, wrb1[...], preferred_element_type=jnp.float32) + brb1[...])
    g = jnp.dot(h, wrb2[...], preferred_element_type=jnp.float32) + brb2[...]
    gamma_ref[...] = _layer_norm(g, lnw[...], lnb[...])


def _dense_pre(x, wfc_t, bfc, wrate_t, wrb1_t, brb1, wrb2_t, brb2, lnw, lnb):
    n, d = x.shape
    blk = 1000
    grid = (n // blk,)
    row_spec = pl.BlockSpec((blk, d), lambda i: (i, 0))
    full = pl.BlockSpec((d, d), lambda i: (0, 0))
    vec = pl.BlockSpec((1, d), lambda i: (0, 0))
    out_sds = jax.ShapeDtypeStruct((n, d), jnp.float32)
    return pl.pallas_call(
        _pre_body,
        grid=grid,
        in_specs=[row_spec, full, vec, full, full, vec, full, vec, vec, vec],
        out_specs=[row_spec, row_spec, row_spec],
        out_shape=[out_sds, out_sds, out_sds],
    )(x, wfc_t, bfc, wrate_t, wrb1_t, brb1, wrb2_t, brb2, lnw, lnb)


def _sc_aggregate(xf0, xf1, row, col):
    n, dh = xf0.shape
    e = row.shape[0]
    info = plsc.get_sparse_core_info()
    nc, ns = info.num_cores, info.num_subcores
    nw = nc * ns
    chunk = 80
    e_per_tile = e // nw
    n_chunks = e_per_tile // chunk
    rows_per_tile = ((n + ns * 8 - 1) // (ns * 8)) * 8
    n_pad = rows_per_tile * ns
    zrows = rows_per_tile // 4
    n_zcopies = rows_per_tile // zrows

    mesh = plsc.VectorSubcoreMesh(core_axis_name="c", subcore_axis_name="s")

    @functools.partial(
        pl.kernel,
        out_type=(
            jax.ShapeDtypeStruct((nc, 2, n_pad, dh), jnp.float32),
            jax.ShapeDtypeStruct((nc, n_pad, 16), jnp.float32),
        ),
        mesh=mesh,
        scratch_types=[
            pltpu.VMEM((chunk,), jnp.int32),
            pltpu.VMEM((chunk,), jnp.int32),
            pltpu.VMEM((chunk, dh), jnp.float32),
            pltpu.VMEM((chunk, 16), jnp.float32),
            pltpu.VMEM((zrows, dh), jnp.float32),
            pltpu.VMEM((zrows, 16), jnp.float32),
            pltpu.VMEM_SHARED((n_pad, dh), jnp.float32),
            pltpu.VMEM_SHARED((n_pad, 16), jnp.float32),
            pltpu.SemaphoreType.DMA,
        ],
        compiler_params=pltpu.CompilerParams(use_tc_tiling_on_sc=False),
    )
    def agg_kernel(xf0_hbm, xf1_hbm, row_hbm, col_hbm, s_out, cnt_out,
                   ridx, cidx, rows, ones, zs, zc, s_sh, c_sh, sem):
        c = lax.axis_index("c")
        s = lax.axis_index("s")
        wid = c * ns + s
        z16 = jnp.zeros((16,), jnp.float32)
        one16 = jnp.ones((16,), jnp.float32)

        def init_z(i, carry):
            for k in range(dh // 16):
                zs[i, pl.ds(k * 16, 16)] = z16
            zc[i] = z16
            return carry

        lax.fori_loop(0, zrows, init_z, 0)

        def init_ones(i, carry):
            ones[i] = one16
            return carry

        lax.fori_loop(0, chunk, init_ones, 0)

        base_rows = s * rows_per_tile
        e0 = wid * e_per_tile

        for h, xfh in ((0, xf0_hbm), (1, xf1_hbm)):
            def zero_copy(i, carry):
                pltpu.sync_copy(zs,
                                s_sh.at[pl.ds(base_rows + i * zrows, zrows)])
                if h == 0:
                    pltpu.sync_copy(
                        zc, c_sh.at[pl.ds(base_rows + i * zrows, zrows)])
                return carry

            lax.fori_loop(0, n_zcopies, zero_copy, 0)
            plsc.subcore_barrier()

            def body(j, carry):
                base = e0 + j * chunk
                pltpu.sync_copy(row_hbm.at[pl.ds(base, chunk)], ridx)
                pltpu.sync_copy(col_hbm.at[pl.ds(base, chunk)], cidx)
                pltpu.async_copy(xfh.at[cidx], rows, sem).wait()
                pltpu.sync_copy(rows, s_sh.at[ridx], add=True)
                if h == 0:
                    pltpu.sync_copy(ones, c_sh.at[ridx], add=True)
                return carry

            lax.fori_loop(0, n_chunks, body, 0)
            plsc.subcore_barrier()

            pltpu.sync_copy(s_sh.at[pl.ds(base_rows, rows_per_tile)],
                            s_out.at[c, h, pl.ds(base_rows, rows_per_tile)])
            if h == 0:
                pltpu.sync_copy(c_sh.at[pl.ds(base_rows, rows_per_tile)],
                                cnt_out.at[c, pl.ds(base_rows, rows_per_tile)])

    return agg_kernel(xf0, xf1, row, col)


def _post_body(xf_ref, rate_ref, gamma_ref, s_ref, cnt_ref, deg_ref, lnw, lnb,
               out_ref):
    xf = xf_ref[...]
    rate = rate_ref[...]
    ssum = jnp.concatenate(
        [s_ref[0, 0] + s_ref[1, 0], s_ref[0, 1] + s_ref[1, 1]], axis=-1)
    cnt = cnt_ref[0, :, 0:1] + cnt_ref[1, :, 0:1]
    agg = cnt * xf + ssum
    denom = 1.0 + rate * deg_ref[...] + EPS
    out = (rate * agg + gamma_ref[...]) / denom - xf
    out_ref[...] = _layer_norm(out, lnw[...], lnb[...])


def _dense_post(xf, rate, gamma, s_part, cnt_part, deg_col, lnw, lnb):
    n, d = xf.shape
    blk = 1000
    grid = (n // blk,)
    row_spec = pl.BlockSpec((blk, d), lambda i: (i, 0))
    s_spec = pl.BlockSpec((2, 2, blk, d // 2), lambda i: (0, 0, i, 0))
    cnt_spec = pl.BlockSpec((2, blk, 16), lambda i: (0, i, 0))
    deg_spec = pl.BlockSpec((blk, 1), lambda i: (i, 0))
    vec = pl.BlockSpec((1, d), lambda i: (0, 0))
    return pl.pallas_call(
        _post_body,
        grid=grid,
        in_specs=[row_spec, row_spec, row_spec, s_spec, cnt_spec, deg_spec,
                  vec, vec],
        out_specs=row_spec,
        out_shape=jax.ShapeDtypeStruct((n, d), jnp.float32),
    )(xf, rate, gamma, s_part, cnt_part, deg_col, lnw, lnb)


def kernel(x, edge_index, degree, W_fc, b_fc, W_rate, W_rb1, b_rb1, W_rb2,
           b_rb2, ln_rb_w, ln_rb_b, ln_w, ln_b):
    n, d = x.shape
    row = edge_index[0]
    col = edge_index[1]
    xf, rate, gamma = _dense_pre(
        x, W_fc.T, b_fc.reshape(1, d), W_rate.T, W_rb1.T, b_rb1.reshape(1, d),
        W_rb2.T, b_rb2.reshape(1, d), ln_rb_w.reshape(1, d),
        ln_rb_b.reshape(1, d))
    s_part, cnt_part = _sc_aggregate(
        xf[:, : d // 2], xf[:, d // 2:], row, col)
    return _dense_post(xf, rate, gamma, s_part, cnt_part,
                       degree.reshape(n, 1), ln_w.reshape(1, d),
                       ln_b.reshape(1, d))
